# baseline (speedup 1.0000x reference)
import functools

import jax
import jax.numpy as jnp
from jax import lax
from jax.experimental import pallas as pl
from jax.experimental.pallas import tpu as pltpu
from jax.experimental.pallas import tpu_sc as plsc

N = 10000
E = 320000
D_IN = 128
DH = 64

NC = 2
NS = 16
NW = NC * NS
CHUNK = 128
NCHUNK = 80
EPT = NCHUNK * CHUNK
EP = EPT * NW
NP = 12288
NPE = 10240
RPT_E = NPE // NS
EPS = E // NS
COLS_PER_TILE = NP // NW

_mesh = lambda: plsc.VectorSubcoreMesh(
    core_axis_name="c", subcore_axis_name="s", num_cores=NC, num_subcores=NS)

_SC_PARAMS = pltpu.CompilerParams(needs_layout_passes=False)
_SC_PARAMS_NT = pltpu.CompilerParams(needs_layout_passes=False,
                                     use_tc_tiling_on_sc=False)

_Z16 = lambda: jnp.zeros((16,), jnp.float32)


def _rsqrt16(d):
    y = plsc.bitcast(jnp.int32(0x5F3759DF) - (plsc.bitcast(d, jnp.int32) >> 1),
                     jnp.float32)
    for _ in range(3):
        y = y * (1.5 - 0.5 * d * y * y)
    return y


@functools.partial(
    pl.kernel,
    out_type=[jax.ShapeDtypeStruct((NC, NPE, DH), jnp.float32),
              jax.ShapeDtypeStruct((NPE,), jnp.float32)],
    mesh=_mesh(),
    scratch_types=[pltpu.VMEM((NCHUNK, CHUNK), jnp.int32),
                   pltpu.VMEM((NCHUNK, CHUNK), jnp.int32),
                   pltpu.VMEM((CHUNK, DH), jnp.float32),
                   pltpu.VMEM((CHUNK, DH), jnp.float32),
                   pltpu.VMEM((CHUNK,), jnp.float32),
                   pltpu.VMEM((RPT_E,), jnp.float32),
                   pltpu.VMEM((RPT_E + 16,), jnp.float32),
                   pltpu.VMEM_SHARED((NPE,), jnp.float32),
                   pltpu.VMEM_SHARED((NPE, DH), jnp.float32),
                   pltpu.VMEM_SHARED((NPE, DH), jnp.float32),
                   pltpu.SemaphoreType.DMA,
                   pltpu.SemaphoreType.DMA],
    compiler_params=_SC_PARAMS_NT,
)
def _edge64_call(src_hbm, dst_hbm, h_hbm, out_hbm, dinv_hbm,
                 sidx_v, didx_v, rows_a, rows_b, ones_v, zer_v, dv_v,
                 deg_sp, acc_sh, m_sp, sem_a, sem_b):
    cid = lax.axis_index("c")
    sid = lax.axis_index("s")
    wid = sid * NC + cid
    rowbase = sid * RPT_E
    cp_s = pltpu.async_copy(src_hbm.at[wid], sidx_v, sem_a)
    z = _Z16()
    one = jnp.ones((16,), jnp.float32)
    for c8 in range(CHUNK // 16):
        ones_v[pl.ds(c8 * 16, 16)] = one

    def zzer(j, carry):
        zer_v[pl.ds(j * 16, 16)] = z
        return carry
    lax.fori_loop(0, RPT_E // 16, zzer, 0)

    def zrow(r, carry):
        for c4 in range(DH // 16):
            rows_a[r, pl.ds(c4 * 16, 16)] = z
        return carry
    lax.fori_loop(0, CHUNK, zrow, 0)

    pltpu.sync_copy(zer_v, deg_sp.at[pl.ds(rowbase, RPT_E)])

    def zacc(k, carry):
        pltpu.sync_copy(
            rows_a, acc_sh.at[pl.ds(rowbase + k * CHUNK, CHUNK)])
        return carry
    lax.fori_loop(0, RPT_E // CHUNK, zacc, 0)
    cp_s.wait()
    plsc.subcore_barrier()

    def hist(r, carry):
        pltpu.sync_copy(dst_hbm.at[2 * sid + r], didx_v)

        def hadd(c, c2):
            pltpu.sync_copy(ones_v, deg_sp.at[didx_v.at[c]], add=True)
            return c2
        lax.fori_loop(0, NCHUNK, hadd, 0)
        return carry
    lax.fori_loop(0, 2, hist, 0)
    plsc.subcore_barrier()

    pltpu.sync_copy(deg_sp.at[pl.ds(rowbase, RPT_E)],
                    dv_v.at[pl.ds(0, RPT_E)])
    lane = lax.iota(jnp.int32, 16)

    def comb(k, carry):
        sl = pl.ds(k * 16, 16)
        d = dv_v[sl] + 1.0
        row = rowbase + k * 16 + lane
        dv_v[sl] = jnp.where(row < N, _rsqrt16(d), 0.0)
        return carry
    lax.fori_loop(0, RPT_E // 16, comb, 0)

    @pl.when(cid == 0)
    def _():
        pltpu.sync_copy(dv_v.at[pl.ds(0, RPT_E)],
                        dinv_hbm.at[pl.ds(rowbase, RPT_E)])

    cp_d = pltpu.async_copy(dst_hbm.at[wid], didx_v, sem_b)

    def stg(k, carry):
        roff = rowbase + k * CHUNK
        pltpu.sync_copy(h_hbm.at[pl.ds(roff, CHUNK)], rows_b)

        def scale(r, c2):
            s = dv_v[pl.ds(k * CHUNK + r, 16)][0]
            for c4 in range(DH // 16):
                sl = pl.ds(c4 * 16, 16)
                rows_b[r, sl] = s * rows_b[r, sl]
            return c2
        lax.fori_loop(0, CHUNK, scale, 0)
        pltpu.sync_copy(rows_b, m_sp.at[pl.ds(roff, CHUNK)])
        return carry
    lax.fori_loop(0, RPT_E // CHUNK, stg, 0)
    cp_d.wait()
    plsc.subcore_barrier()

    pltpu.async_copy(m_sp.at[sidx_v.at[0]], rows_a, sem_a)

    def body(i, carry):
        c0 = 2 * i
        c1 = c0 + 1
        pltpu.make_async_copy(m_sp.at[sidx_v.at[c0]], rows_a, sem_a).wait()
        pltpu.async_copy(m_sp.at[sidx_v.at[c1]], rows_b, sem_b)
        pltpu.sync_copy(rows_a, acc_sh.at[didx_v.at[c0]], add=True)
        pltpu.make_async_copy(m_sp.at[sidx_v.at[c1]], rows_b, sem_b).wait()

        @pl.when(i < NCHUNK // 2 - 1)
        def _():
            pltpu.async_copy(m_sp.at[sidx_v.at[c0 + 2]], rows_a, sem_a)
        pltpu.sync_copy(rows_b, acc_sh.at[didx_v.at[c1]], add=True)
        return carry
    lax.fori_loop(0, NCHUNK // 2, body, 0)
    plsc.subcore_barrier()

    def out(k, carry):
        roff = sid * RPT_E + k * CHUNK
        pltpu.sync_copy(acc_sh.at[pl.ds(roff, CHUNK)],
                        out_hbm.at[cid, pl.ds(roff, CHUNK)])
        return carry
    lax.fori_loop(0, RPT_E // CHUNK, out, 0)


@functools.partial(
    pl.kernel,
    out_type=jax.ShapeDtypeStruct((NP,), jnp.float32),
    mesh=_mesh(),
    scratch_types=[pltpu.VMEM((NP,), jnp.float32),
                   pltpu.VMEM((EPS,), jnp.int32),
                   pltpu.VMEM((EPS,), jnp.int32),
                   pltpu.VMEM((NP,), jnp.float32),
                   pltpu.VMEM((NS, COLS_PER_TILE), jnp.float32),
                   pltpu.VMEM((COLS_PER_TILE,), jnp.float32),
                   pltpu.VMEM((COLS_PER_TILE,), jnp.float32),
                   pltpu.VMEM((16,), jnp.float32),
                   pltpu.VMEM_SHARED((NS, NP), jnp.float32),
                   pltpu.SemaphoreType.DMA,
                   pltpu.SemaphoreType.DMA,
                   pltpu.SemaphoreType.DMA],
    compiler_params=_SC_PARAMS,
)
def _edge1f_call(src_hbm, dst_hbm, t_hbm, dinv_hbm, b2_hbm, out_hbm,
                 t_v, sidx_v, didx_v, acc_v, blk_v, dv_v, res_v, b2_v,
                 stage_sp, sem_a, sem_b, sem_c):
    cid = lax.axis_index("c")
    sid = lax.axis_index("s")
    cp_t = pltpu.async_copy(t_hbm, t_v, sem_a)
    cp_s = pltpu.async_copy(src_hbm.at[pl.ds(sid * EPS, EPS)], sidx_v, sem_b)
    cp_d = pltpu.async_copy(dst_hbm.at[pl.ds(sid * EPS, EPS)], didx_v, sem_c)
    pltpu.sync_copy(b2_hbm, b2_v)
    z = _Z16()

    def zero(j, carry):
        acc_v[pl.ds(j * 16, 16)] = z
        return carry
    lax.fori_loop(0, NP // 16, zero, 0)
    cp_t.wait()
    cp_s.wait()
    cp_d.wait()

    def body(j, carry):
        sl = pl.ds(j * 16, 16)
        sv = sidx_v[sl]
        dv = didx_v[sl]
        vals = plsc.load_gather(t_v, [sv])
        plsc.addupdate_scatter(acc_v, [dv], vals)
        return carry
    lax.fori_loop(0, EPS // 16, body, 0)

    pltpu.sync_copy(acc_v, stage_sp.at[sid])
    plsc.subcore_barrier()

    colbase = cid * (NP // NC) + sid * COLS_PER_TILE
    pltpu.sync_copy(stage_sp.at[:, pl.ds(colbase, COLS_PER_TILE)], blk_v)
    pltpu.sync_copy(dinv_hbm.at[pl.ds(colbase, COLS_PER_TILE)], dv_v)
    b2 = b2_v[pl.ds(0, 16)]

    def comb(k, carry):
        sl = pl.ds(k * 16, 16)
        es = blk_v[0, sl]
        for s in range(1, NS):
            es = es + blk_v[s, sl]
        tt = t_v[pl.ds(colbase + k * 16, 16)]
        res_v[sl] = dv_v[sl] * (es + tt) + b2
        return carry
    lax.fori_loop(0, COLS_PER_TILE // 16, comb, 0)
    pltpu.sync_copy(res_v, out_hbm.at[pl.ds(colbase, COLS_PER_TILE)])


def _prep_body(x_ref, w1_ref, h_ref):
    h = jnp.dot(x_ref[...], w1_ref[...],
                preferred_element_type=jnp.float32,
                precision=lax.Precision.HIGHEST)
    h_ref[...] = jnp.concatenate(
        [h, jnp.zeros((NPE - N, DH), jnp.float32)], axis=0)


_prep_call = pl.pallas_call(
    _prep_body,
    out_shape=jax.ShapeDtypeStruct((NPE, DH), jnp.float32),
)


def _mid_body(accp_ref, h_ref, dinv_ref, b1_ref, w2r_ref, t_ref, dinvp_ref):
    df = dinv_ref[...]
    m = df * h_ref[...]
    acc = accp_ref[0] + accp_ref[1] + m
    a = jnp.maximum(df * acc + b1_ref[...], 0.0)
    s = jnp.sum(a * w2r_ref[...], axis=1, keepdims=True)
    zpad = jnp.zeros((NP - NPE, 1), jnp.float32)
    t_ref[...] = jnp.concatenate([df * s, zpad], axis=0)
    dinvp_ref[...] = jnp.concatenate([df, zpad], axis=0)


_mid_call = pl.pallas_call(
    _mid_body,
    out_shape=[jax.ShapeDtypeStruct((NP, 1), jnp.float32),
               jax.ShapeDtypeStruct((NP, 1), jnp.float32)],
)


def kernel(x, edge_index, node_id, W1, b1, W2, b2):
    src = edge_index[0]
    dst = edge_index[1]
    pad = jnp.full((EP - E,), N, jnp.int32)
    src3 = jnp.concatenate([src, pad]).reshape(NW, NCHUNK, CHUNK)
    dst3 = jnp.concatenate([dst, pad]).reshape(NW, NCHUNK, CHUNK)

    h = _prep_call(x, W1)
    accp, dinv = _edge64_call(src3, dst3, h)
    t, dinvp = _mid_call(accp, h, dinv.reshape(NPE, 1),
                         b1.reshape(1, DH), W2.reshape(1, DH))
    out = _edge1f_call(src, dst, t.reshape(NP), dinvp.reshape(NP),
                       jnp.broadcast_to(b2, (16,)))
    return out[:N]


# speedup vs baseline: 1.0671x; 1.0671x over previous
import functools

import jax
import jax.numpy as jnp
from jax import lax
from jax.experimental import pallas as pl
from jax.experimental.pallas import tpu as pltpu
from jax.experimental.pallas import tpu_sc as plsc

N = 10000
E = 320000
D_IN = 128
DH = 64

NC = 2
NS = 16
NW = NC * NS
CHUNK = 128
NCHUNK = 80
EPT = NCHUNK * CHUNK
EP = EPT * NW
NP = 12288
NPE = 10240
RPT_E = NPE // NS
EPS = E // NS
COLS_PER_TILE = NP // NW

_mesh = lambda: plsc.VectorSubcoreMesh(
    core_axis_name="c", subcore_axis_name="s", num_cores=NC, num_subcores=NS)

_SC_PARAMS = pltpu.CompilerParams(needs_layout_passes=False)
_SC_PARAMS_NT = pltpu.CompilerParams(needs_layout_passes=False,
                                     use_tc_tiling_on_sc=False)

_Z16 = lambda: jnp.zeros((16,), jnp.float32)


def _rsqrt16(d):
    y = plsc.bitcast(jnp.int32(0x5F3759DF) - (plsc.bitcast(d, jnp.int32) >> 1),
                     jnp.float32)
    for _ in range(3):
        y = y * (1.5 - 0.5 * d * y * y)
    return y


@functools.partial(
    pl.kernel,
    out_type=[jax.ShapeDtypeStruct((NC, NPE, DH), jnp.float32),
              jax.ShapeDtypeStruct((NPE,), jnp.float32)],
    mesh=_mesh(),
    scratch_types=[pltpu.VMEM((EPT,), jnp.int32),
                   pltpu.VMEM((NCHUNK, CHUNK), jnp.int32),
                   pltpu.VMEM((CHUNK, DH), jnp.float32),
                   pltpu.VMEM((CHUNK, DH), jnp.float32),
                   pltpu.VMEM((CHUNK,), jnp.float32),
                   pltpu.VMEM((RPT_E,), jnp.float32),
                   pltpu.VMEM((RPT_E + 16,), jnp.float32),
                   pltpu.VMEM_SHARED((NPE,), jnp.float32),
                   pltpu.VMEM_SHARED((NPE, DH), jnp.float32),
                   pltpu.VMEM_SHARED((NPE, DH), jnp.float32),
                   pltpu.SemaphoreType.DMA,
                   pltpu.SemaphoreType.DMA,
                   pltpu.SemaphoreType.DMA],
    compiler_params=_SC_PARAMS_NT,
)
def _edge64_call(src_hbm, dst_hbm, h_hbm, out_hbm, dinv_hbm,
                 sidx_v, didx_v, rows_a, rows_b, ones_v, zer_v, dv_v,
                 deg_sp, acc_sh, m_sp, sem_a, sem_b, sem_h):
    cid = lax.axis_index("c")
    sid = lax.axis_index("s")
    wid = sid * NC + cid
    rowbase = sid * RPT_E
    cp_s = pltpu.async_copy(src_hbm.at[pl.ds(wid * EPT, EPT)], sidx_v, sem_a)
    z = _Z16()
    one = jnp.ones((16,), jnp.float32)
    for c8 in range(CHUNK // 16):
        ones_v[pl.ds(c8 * 16, 16)] = one

    def zzer(j, carry):
        zer_v[pl.ds(j * 16, 16)] = z
        return carry
    lax.fori_loop(0, RPT_E // 16, zzer, 0)

    def zrow(r, carry):
        for c4 in range(DH // 16):
            rows_a[r, pl.ds(c4 * 16, 16)] = z
        return carry
    lax.fori_loop(0, CHUNK, zrow, 0)

    pltpu.async_copy(zer_v, deg_sp.at[pl.ds(rowbase, RPT_E)], sem_b)

    def zacc(k, carry):
        pltpu.async_copy(
            rows_a, acc_sh.at[pl.ds(rowbase + k * CHUNK, CHUNK)], sem_b)
        return carry
    lax.fori_loop(0, RPT_E // CHUNK, zacc, 0)
    cp_s.wait()
    pltpu.make_async_copy(zer_v, deg_sp.at[pl.ds(rowbase, RPT_E)],
                          sem_b).wait()

    def zacc_drain(k, carry):
        pltpu.make_async_copy(
            rows_a, acc_sh.at[pl.ds(rowbase + k * CHUNK, CHUNK)],
            sem_b).wait()
        return carry
    lax.fori_loop(0, RPT_E // CHUNK, zacc_drain, 0)
    plsc.subcore_barrier()

    def hist(r, carry):
        pltpu.sync_copy(dst_hbm.at[2 * sid + r], didx_v)

        def hadd(c, c2):
            pltpu.async_copy(ones_v, deg_sp.at[didx_v.at[c]], sem_h,
                             add=True)
            return c2
        lax.fori_loop(0, NCHUNK, hadd, 0)

        def hdrain(c, c2):
            pltpu.make_async_copy(ones_v, deg_sp.at[didx_v.at[0]],
                                  sem_h).wait()
            return c2
        lax.fori_loop(0, NCHUNK, hdrain, 0)
        return carry
    lax.fori_loop(0, 2, hist, 0)
    plsc.subcore_barrier()

    pltpu.sync_copy(deg_sp.at[pl.ds(rowbase, RPT_E)],
                    dv_v.at[pl.ds(0, RPT_E)])
    lane = lax.iota(jnp.int32, 16)

    def comb(k, carry):
        sl = pl.ds(k * 16, 16)
        d = dv_v[sl] + 1.0
        row = rowbase + k * 16 + lane
        dv_v[sl] = jnp.where(row < N, _rsqrt16(d), 0.0)
        return carry
    lax.fori_loop(0, RPT_E // 16, comb, 0)

    @pl.when(cid == 0)
    def _():
        pltpu.sync_copy(dv_v.at[pl.ds(0, RPT_E)],
                        dinv_hbm.at[pl.ds(rowbase, RPT_E)])

    cp_d = pltpu.async_copy(dst_hbm.at[wid], didx_v, sem_b)

    bufs = (rows_a, rows_b)
    nstg = RPT_E // CHUNK
    d_in = [None, None]
    d_out = [None, None]
    d_in[0] = pltpu.async_copy(h_hbm.at[pl.ds(rowbase, CHUNK)], bufs[0],
                               sem_a)
    for k in range(nstg):
        b = k % 2
        roff = rowbase + k * CHUNK
        d_in[b].wait()
        if k + 1 < nstg:
            b2 = 1 - b
            if d_out[b2] is not None:
                d_out[b2].wait()
            d_in[b2] = pltpu.async_copy(
                h_hbm.at[pl.ds(rowbase + (k + 1) * CHUNK, CHUNK)],
                bufs[b2], sem_a)
        buf = bufs[b]

        def scale(r, c2, k=k, buf=buf):
            s = dv_v[pl.ds(k * CHUNK + r, 16)][0]
            for c4 in range(DH // 16):
                sl = pl.ds(c4 * 16, 16)
                buf[r, sl] = s * buf[r, sl]
            return c2
        lax.fori_loop(0, CHUNK, scale, 0)
        d_out[b] = pltpu.async_copy(buf, m_sp.at[pl.ds(roff, CHUNK)], sem_h)
    d_out[0].wait()
    d_out[1].wait()
    cp_d.wait()
    plsc.subcore_barrier()

    def _sidx(c):
        return sidx_v.at[pl.ds(c * CHUNK, CHUNK)]

    pltpu.async_copy(m_sp.at[_sidx(0)], rows_a, sem_a)

    def body(i, carry):
        c0 = 2 * i
        c1 = c0 + 1
        pltpu.make_async_copy(m_sp.at[_sidx(c0)], rows_a, sem_a).wait()
        pltpu.async_copy(m_sp.at[_sidx(c1)], rows_b, sem_b)
        pltpu.sync_copy(rows_a, acc_sh.at[didx_v.at[c0]], add=True)
        pltpu.make_async_copy(m_sp.at[_sidx(c1)], rows_b, sem_b).wait()

        @pl.when(i < NCHUNK // 2 - 1)
        def _():
            pltpu.async_copy(m_sp.at[_sidx(c0 + 2)], rows_a, sem_a)
        pltpu.sync_copy(rows_b, acc_sh.at[didx_v.at[c1]], add=True)
        return carry
    lax.fori_loop(0, NCHUNK // 2, body, 0)
    plsc.subcore_barrier()

    def out(k, carry):
        roff = sid * RPT_E + k * CHUNK
        pltpu.async_copy(acc_sh.at[pl.ds(roff, CHUNK)],
                        out_hbm.at[cid, pl.ds(roff, CHUNK)], sem_a)
        return carry
    lax.fori_loop(0, RPT_E // CHUNK, out, 0)

    def out_drain(k, carry):
        roff = sid * RPT_E + k * CHUNK
        pltpu.make_async_copy(acc_sh.at[pl.ds(roff, CHUNK)],
                              out_hbm.at[cid, pl.ds(roff, CHUNK)],
                              sem_a).wait()
        return carry
    lax.fori_loop(0, RPT_E // CHUNK, out_drain, 0)


@functools.partial(
    pl.kernel,
    out_type=jax.ShapeDtypeStruct((NP,), jnp.float32),
    mesh=_mesh(),
    scratch_types=[pltpu.VMEM((NP,), jnp.float32),
                   pltpu.VMEM((EPS,), jnp.int32),
                   pltpu.VMEM((EPS,), jnp.int32),
                   pltpu.VMEM((NP,), jnp.float32),
                   pltpu.VMEM((NS, COLS_PER_TILE), jnp.float32),
                   pltpu.VMEM((COLS_PER_TILE,), jnp.float32),
                   pltpu.VMEM((COLS_PER_TILE,), jnp.float32),
                   pltpu.VMEM((16,), jnp.float32),
                   pltpu.VMEM_SHARED((NS, NP), jnp.float32),
                   pltpu.SemaphoreType.DMA,
                   pltpu.SemaphoreType.DMA,
                   pltpu.SemaphoreType.DMA],
    compiler_params=_SC_PARAMS,
)
def _edge1f_call(src_hbm, dst_hbm, t_hbm, dinv_hbm, b2_hbm, out_hbm,
                 t_v, sidx_v, didx_v, acc_v, blk_v, dv_v, res_v, b2_v,
                 stage_sp, sem_a, sem_b, sem_c):
    cid = lax.axis_index("c")
    sid = lax.axis_index("s")
    cp_t = pltpu.async_copy(t_hbm, t_v, sem_a)
    cp_s = pltpu.async_copy(src_hbm.at[pl.ds(sid * EPS, EPS)], sidx_v, sem_b)
    cp_d = pltpu.async_copy(dst_hbm.at[pl.ds(sid * EPS, EPS)], didx_v, sem_c)
    pltpu.sync_copy(b2_hbm, b2_v)
    z = _Z16()

    def zero(j, carry):
        acc_v[pl.ds(j * 16, 16)] = z
        return carry
    lax.fori_loop(0, NP // 16, zero, 0)
    cp_t.wait()
    cp_s.wait()
    cp_d.wait()

    def body(j, carry):
        sl = pl.ds(j * 16, 16)
        sv = sidx_v[sl]
        dv = didx_v[sl]
        vals = plsc.load_gather(t_v, [sv])
        plsc.addupdate_scatter(acc_v, [dv], vals)
        return carry
    lax.fori_loop(0, EPS // 16, body, 0)

    pltpu.sync_copy(acc_v, stage_sp.at[sid])
    plsc.subcore_barrier()

    colbase = cid * (NP // NC) + sid * COLS_PER_TILE
    pltpu.sync_copy(stage_sp.at[:, pl.ds(colbase, COLS_PER_TILE)], blk_v)
    pltpu.sync_copy(dinv_hbm.at[pl.ds(colbase, COLS_PER_TILE)], dv_v)
    b2 = b2_v[pl.ds(0, 16)]

    def comb(k, carry):
        sl = pl.ds(k * 16, 16)
        es = blk_v[0, sl]
        for s in range(1, NS):
            es = es + blk_v[s, sl]
        tt = t_v[pl.ds(colbase + k * 16, 16)]
        res_v[sl] = dv_v[sl] * (es + tt) + b2
        return carry
    lax.fori_loop(0, COLS_PER_TILE // 16, comb, 0)
    pltpu.sync_copy(res_v, out_hbm.at[pl.ds(colbase, COLS_PER_TILE)])


def _prep_body(x_ref, w1_ref, h_ref):
    h = jnp.dot(x_ref[...], w1_ref[...],
                preferred_element_type=jnp.float32)
    h_ref[...] = jnp.concatenate(
        [h, jnp.zeros((NPE - N, DH), jnp.float32)], axis=0)


_prep_call = pl.pallas_call(
    _prep_body,
    out_shape=jax.ShapeDtypeStruct((NPE, DH), jnp.float32),
)


def _mid_body(accp_ref, h_ref, dinv_ref, b1_ref, w2r_ref, t_ref, dinvp_ref):
    df = dinv_ref[...]
    m = df * h_ref[...]
    acc = accp_ref[0] + accp_ref[1] + m
    a = jnp.maximum(df * acc + b1_ref[...], 0.0)
    s = jnp.sum(a * w2r_ref[...], axis=1, keepdims=True)
    zpad = jnp.zeros((NP - NPE, 1), jnp.float32)
    t_ref[...] = jnp.concatenate([df * s, zpad], axis=0)
    dinvp_ref[...] = jnp.concatenate([df, zpad], axis=0)


_mid_call = pl.pallas_call(
    _mid_body,
    out_shape=[jax.ShapeDtypeStruct((NP, 1), jnp.float32),
               jax.ShapeDtypeStruct((NP, 1), jnp.float32)],
)


def kernel(x, edge_index, node_id, W1, b1, W2, b2):
    src = edge_index[0]
    dst = edge_index[1]
    pad = jnp.full((EP - E,), N, jnp.int32)
    srcp = jnp.concatenate([src, pad])
    dst3 = jnp.concatenate([dst, pad]).reshape(NW, NCHUNK, CHUNK)

    h = _prep_call(x, W1)
    accp, dinv = _edge64_call(srcp, dst3, h)
    t, dinvp = _mid_call(accp, h, dinv.reshape(NPE, 1),
                         b1.reshape(1, DH), W2.reshape(1, DH))
    out = _edge1f_call(src, dst, t.reshape(NP), dinvp.reshape(NP),
                       jnp.broadcast_to(b2, (16,)))
    return out[:N]
